# baseline (speedup 1.0000x reference)
import jax
import jax.numpy as jnp
from jax.experimental import pallas as pl
from jax.experimental.pallas import tpu as pltpu


def _body(nblk, C, V):
    def body(dist_ref, act_ref, lpw_ref, y_ref,
             fp_ref, fd_ref, lq_ref, loss_ref,
             sum_acc, val_acc):
        pid = pl.program_id(0)
        T, B = act_ref.shape

        @pl.when(pid == 0)
        def _():
            sum_acc[...] = jnp.zeros_like(sum_acc)
            val_acc[...] = jnp.zeros_like(val_acc)

        x = dist_ref[...]
        act = act_ref[...]
        base = pid * C
        lane = jax.lax.broadcasted_iota(jnp.int32, x.shape, 2) + base
        hit = lane == act[..., None]

        @pl.when(pid < nblk - 1)
        def _():
            sum_acc[...] += x.sum(-1)
            val_acc[...] += jnp.where(hit, x, 0.0).sum(-1)

        @pl.when(pid == nblk - 1)
        def _():
            xm = jnp.where(lane < V, x, 0.0)
            sum_acc[...] += xm.sum(-1)
            val_acc[...] += jnp.where(hit, xm, 0.0).sum(-1)

            probs = val_acc[...] / sum_acc[...]
            fp_ref[...] = probs.T
            fd_ref[...] = probs[T - 1:T, :]
            lq = jnp.log(probs).sum(0, keepdims=True)
            lq_ref[...] = lq
            lp = (1.0 - y_ref[...]) * jnp.log(jnp.float32(1e-8)) + lpw_ref[...]
            d = lq - lp
            loss_ref[...] = jnp.mean(d * d).reshape(1, 1)

    return body


def kernel(distributions, actions, log_p_world, y):
    T, B, V = distributions.shape
    C = 4096
    nblk = pl.cdiv(V, C)
    f32 = jnp.float32

    fp, fd, lq, loss = pl.pallas_call(
        _body(nblk, C, V),
        grid=(nblk,),
        in_specs=[
            pl.BlockSpec((T, B, C), lambda i: (0, 0, i)),
            pl.BlockSpec((T, B), lambda i: (0, 0)),
            pl.BlockSpec((1, B), lambda i: (0, 0)),
            pl.BlockSpec((1, B), lambda i: (0, 0)),
        ],
        out_specs=[
            pl.BlockSpec((B, T), lambda i: (0, 0)),
            pl.BlockSpec((1, B), lambda i: (0, 0)),
            pl.BlockSpec((1, B), lambda i: (0, 0)),
            pl.BlockSpec((1, 1), lambda i: (0, 0)),
        ],
        out_shape=[
            jax.ShapeDtypeStruct((B, T), f32),
            jax.ShapeDtypeStruct((1, B), f32),
            jax.ShapeDtypeStruct((1, B), f32),
            jax.ShapeDtypeStruct((1, 1), f32),
        ],
        scratch_shapes=[
            pltpu.VMEM((T, B), f32),
            pltpu.VMEM((T, B), f32),
        ],
        compiler_params=pltpu.CompilerParams(
            dimension_semantics=("arbitrary",),
        ),
    )(distributions, actions,
      log_p_world.reshape(1, B), y.reshape(1, B))

    return fp, fd.reshape(B), lq.reshape(B), loss[0, 0]


# speedup vs baseline: 3.8054x; 3.8054x over previous
import jax
import jax.numpy as jnp
from jax.experimental import pallas as pl
from jax.experimental.pallas import tpu as pltpu


def _body(nblk, C):
    def body(dist_ref, act_ref, lpw_ref, y_ref,
             fp_ref, fd_ref, lq_ref, loss_ref,
             sum_acc, val_acc):
        pid = pl.program_id(0)
        T, _, B = dist_ref.shape

        @pl.when(pid == 0)
        def _():
            sum_acc[...] = jnp.zeros_like(sum_acc)
            val_acc[...] = jnp.zeros_like(val_acc)

        x = dist_ref[...]
        act = act_ref[...]
        vidx = jax.lax.broadcasted_iota(jnp.int32, x.shape, 1) + pid * C
        hit = vidx == act[:, None, :]
        sum_acc[...] += x.sum(1)
        val_acc[...] += jnp.where(hit, x, 0.0).sum(1)

        @pl.when(pid == nblk - 1)
        def _():
            probs = val_acc[...] / sum_acc[...]
            fp_ref[...] = probs.T
            fd_ref[...] = probs[T - 1:T, :]
            lq = jnp.log(probs).sum(0, keepdims=True)
            lq_ref[...] = lq
            lp = (1.0 - y_ref[...]) * jnp.log(jnp.float32(1e-8)) + lpw_ref[...]
            d = lq - lp
            loss_ref[...] = jnp.mean(d * d).reshape(1, 1)

    return body


def kernel(distributions, actions, log_p_world, y):
    T, B, V = distributions.shape
    C = 5000
    nblk = V // C
    f32 = jnp.float32

    dvb = jnp.transpose(distributions, (0, 2, 1))

    fp, fd, lq, loss = pl.pallas_call(
        _body(nblk, C),
        grid=(nblk,),
        in_specs=[
            pl.BlockSpec((T, C, B), lambda i: (0, i, 0)),
            pl.BlockSpec((T, B), lambda i: (0, 0)),
            pl.BlockSpec((1, B), lambda i: (0, 0)),
            pl.BlockSpec((1, B), lambda i: (0, 0)),
        ],
        out_specs=[
            pl.BlockSpec((B, T), lambda i: (0, 0)),
            pl.BlockSpec((1, B), lambda i: (0, 0)),
            pl.BlockSpec((1, B), lambda i: (0, 0)),
            pl.BlockSpec((1, 1), lambda i: (0, 0)),
        ],
        out_shape=[
            jax.ShapeDtypeStruct((B, T), f32),
            jax.ShapeDtypeStruct((1, B), f32),
            jax.ShapeDtypeStruct((1, B), f32),
            jax.ShapeDtypeStruct((1, 1), f32),
        ],
        scratch_shapes=[
            pltpu.VMEM((T, B), f32),
            pltpu.VMEM((T, B), f32),
        ],
        compiler_params=pltpu.CompilerParams(
            dimension_semantics=("arbitrary",),
        ),
    )(dvb, actions, log_p_world.reshape(1, B), y.reshape(1, B))

    return fp, fd.reshape(B), lq.reshape(B), loss[0, 0]
